# baseline (speedup 1.0000x reference)
import functools

import jax
import jax.numpy as jnp
from jax import lax
from jax.experimental import pallas as pl
from jax.experimental.pallas import tpu as pltpu
from jax.experimental.pallas import tpu_sc as plsc


@functools.cache
def _build(seq, batch, vocab, d):
    n = seq * batch
    info = plsc.get_sparse_core_info()
    nc = info.num_cores
    nw = nc * info.num_subcores
    ch = 128
    kk = 4
    sb = batch // ch
    nch = n // (nw * ch)
    nblk = nch // kk
    assert n == nw * nch * ch and nblk % 2 == 0 and seq % 8 == 0

    mesh = plsc.VectorSubcoreMesh(core_axis_name="c", subcore_axis_name="s")

    @functools.partial(
        pl.kernel,
        mesh=mesh,
        out_type=jax.ShapeDtypeStruct((seq, batch, d), jnp.float32),
        scratch_types=[
            pltpu.VMEM((nch, ch), jnp.int32),
            pltpu.VMEM((2, kk * ch, d), jnp.float32),
            pltpu.SemaphoreType.DMA,
            pltpu.SemaphoreType.DMA,
            pltpu.SemaphoreType.DMA,
            pltpu.SemaphoreType.DMA,
        ],
        compiler_params=pltpu.CompilerParams(use_tc_tiling_on_sc=False),
    )
    def run(table_hbm, idx_hbm, out_hbm, idx_v, rows_v, g0, g1, o0, o1):
        gsem = (g0, g1)
        osem = (o0, o1)
        wid = lax.axis_index("s") * nc + lax.axis_index("c")
        r0 = wid * nch

        pltpu.sync_copy(idx_hbm.at[wid], idx_v)

        def fire_block(g, s):
            for b in range(kk):
                j = g * kk + b
                pltpu.async_copy(
                    table_hbm.at[idx_v.at[j]],
                    rows_v.at[s, pl.ds(b * ch, ch)],
                    gsem[s],
                )

        def drain_gathers(s):
            for b in range(kk):
                pltpu.make_async_copy(
                    table_hbm.at[idx_v.at[0]],
                    rows_v.at[s, pl.ds(b * ch, ch)],
                    gsem[s],
                ).wait()

        def store_block(g, s):
            for b in range(kk):
                r = r0 + g * kk + b
                sr = 8 * (r // (sb * 8)) + r % 8
                bb = (r // 8) % sb
                pltpu.async_copy(
                    rows_v.at[s, pl.ds(b * ch, ch)],
                    out_hbm.at[sr, pl.ds(bb * ch, ch)],
                    osem[s],
                )

        def wait_out(s):
            for b in range(kk):
                pltpu.make_async_copy(
                    rows_v.at[s, pl.ds(b * ch, ch)],
                    out_hbm.at[0, pl.ds(0, ch)],
                    osem[s],
                ).wait()

        fire_block(0, 0)

        def body(g2, carry):
            for s in range(2):
                g = g2 * 2 + s
                drain_gathers(s)
                store_block(g, s)

                @pl.when(g >= 1)
                def _():
                    wait_out(1 - s)

                @pl.when(g + 1 < nblk)
                def _():
                    fire_block(g + 1, 1 - s)

            return carry

        lax.fori_loop(0, nblk // 2, body, 0)
        wait_out(1)

    return run, nw, nch, ch


def kernel(input_ids, table):
    seq, batch = input_ids.shape
    vocab, d = table.shape
    run, nw, nch, ch = _build(seq, batch, vocab, d)
    idx3 = (
        input_ids.reshape(seq // 8, 8, batch // ch, ch)
        .transpose(0, 2, 1, 3)
        .reshape(nw, nch, ch)
    )
    return run(table, idx3)


# speedup vs baseline: 1.3464x; 1.3464x over previous
import functools

import jax
import jax.numpy as jnp
from jax import lax
from jax.experimental import pallas as pl
from jax.experimental.pallas import tpu as pltpu
from jax.experimental.pallas import tpu_sc as plsc


@functools.cache
def _build(seq, batch, vocab, d):
    n = seq * batch
    info = plsc.get_sparse_core_info()
    nc = info.num_cores
    nw = nc * info.num_subcores
    ch = 128
    dp = 2 * d
    kk = 4
    nch = n // (nw * ch)
    nblk = nch // kk
    assert n == nw * nch * ch and nblk % 2 == 0 and seq % 8 == 0

    mesh = plsc.VectorSubcoreMesh(core_axis_name="c", subcore_axis_name="s")

    @functools.partial(
        pl.kernel,
        mesh=mesh,
        out_type=jax.ShapeDtypeStruct((n, dp), jnp.float32),
        scratch_types=[
            pltpu.VMEM((nch, ch), jnp.int32),
            pltpu.VMEM((2, kk * ch, d), jnp.float32),
            pltpu.SemaphoreType.DMA,
            pltpu.SemaphoreType.DMA,
            pltpu.SemaphoreType.DMA,
            pltpu.SemaphoreType.DMA,
        ],
        compiler_params=pltpu.CompilerParams(use_tc_tiling_on_sc=False),
    )
    def run(table_hbm, idx_hbm, out_hbm, idx_v, rows_v, g0, g1, o0, o1):
        gsem = (g0, g1)
        osem = (o0, o1)
        sb = batch // ch
        wid = lax.axis_index("s") * nc + lax.axis_index("c")
        r0 = wid * nch

        pltpu.sync_copy(idx_hbm.at[wid], idx_v)

        def fire_block(g, s):
            for b in range(kk):
                j = g * kk + b
                pltpu.async_copy(
                    table_hbm.at[idx_v.at[j]],
                    rows_v.at[s, pl.ds(b * ch, ch)],
                    gsem[s],
                )

        def drain_gathers(s):
            for b in range(kk):
                pltpu.make_async_copy(
                    table_hbm.at[idx_v.at[0]],
                    rows_v.at[s, pl.ds(b * ch, ch)],
                    gsem[s],
                ).wait()

        def store_block(g, s):
            for b in range(kk):
                r = r0 + g * kk + b
                sr = 8 * (r // (sb * 8)) + r % 8
                bb = (r // 8) % sb
                pltpu.async_copy(
                    rows_v.at[s, pl.ds(b * ch, ch)],
                    out_hbm.at[pl.ds(sr * batch + bb * ch, ch), pl.ds(0, d)],
                    osem[s],
                )

        def wait_out(s):
            for b in range(kk):
                pltpu.make_async_copy(
                    rows_v.at[s, pl.ds(b * ch, ch)],
                    out_hbm.at[pl.ds(0, ch), pl.ds(0, d)],
                    osem[s],
                ).wait()

        fire_block(0, 0)

        def body(g2, carry):
            for s in range(2):
                g = g2 * 2 + s
                drain_gathers(s)
                store_block(g, s)

                @pl.when(g >= 1)
                def _():
                    wait_out(1 - s)

                @pl.when(g + 1 < nblk)
                def _():
                    fire_block(g + 1, 1 - s)

            return carry

        lax.fori_loop(0, nblk // 2, body, 0)
        wait_out(1)

    return run, nw, nch, ch


def kernel(input_ids, table):
    seq, batch = input_ids.shape
    vocab, d = table.shape
    run, nw, nch, ch = _build(seq, batch, vocab, d)
    idx3 = (
        input_ids.reshape(seq // 8, 8, batch // ch, ch)
        .transpose(0, 2, 1, 3)
        .reshape(nw, nch, ch)
    )
    outp = run(table, idx3)
    return outp.reshape(seq, batch, 2 * d)[:, :, :d]
